# baseline (speedup 1.0000x reference)
import jax
import jax.numpy as jnp
from jax import lax
from jax.experimental import pallas as pl
from jax.experimental.pallas import tpu as pltpu


_NCH = 2


def _som_body(x_hbm, w_hbm, wout_hbm, winner_hbm,
              x_v, w_v, win_v, sem_x, sem_w0, sem_w1, sem_out, sem_win):
    M = w_v.shape[0]
    ch = M // _NCH
    sem_w = (sem_w0, sem_w1)
    cp_w = [
        pltpu.make_async_copy(w_hbm.at[pl.ds(c * ch, ch)],
                              w_v.at[pl.ds(c * ch, ch)], sem_w[c])
        for c in range(_NCH)
    ]
    for c in range(_NCH):
        cp_w[c].start()
    cp_x = pltpu.make_async_copy(x_hbm, x_v, sem_x)
    cp_x.start()
    cp_out = []
    vmin, varg = None, None
    for c in range(_NCH):
        cp_w[c].wait()
        cp = pltpu.make_async_copy(w_v.at[pl.ds(c * ch, ch)],
                                   wout_hbm.at[pl.ds(c * ch, ch)], sem_out)
        cp.start()
        cp_out.append(cp)
        w = w_v[pl.ds(c * ch, ch), :]
        ws = w * -2.0
        wn = jnp.sum(w * w, axis=1, keepdims=True)
        if c == 0:
            cp_x.wait()
        x = x_v[...]
        xwt = lax.dot_general(ws, x, (((1,), (1,)), ((), ())),
                              preferred_element_type=jnp.float32)
        d2t = xwt + wn
        bmin = jnp.min(d2t, axis=0, keepdims=True)
        barg = jnp.argmin(d2t, axis=0).astype(jnp.int32)[None, :] + c * ch
        if c == 0:
            vmin, varg = bmin, barg
        else:
            take = bmin < vmin
            vmin = jnp.where(take, bmin, vmin)
            varg = jnp.where(take, barg, varg)
    win_v[...] = varg
    cp_win = pltpu.make_async_copy(win_v, winner_hbm, sem_win)
    cp_win.start()
    cp_win.wait()
    for cp in cp_out:
        cp.wait()


def kernel(inputs, W):
    B, D = inputs.shape
    M, _ = W.shape
    wout, _winner = pl.pallas_call(
        _som_body,
        in_specs=[
            pl.BlockSpec(memory_space=pltpu.MemorySpace.HBM),
            pl.BlockSpec(memory_space=pltpu.MemorySpace.HBM),
        ],
        out_specs=[
            pl.BlockSpec(memory_space=pltpu.MemorySpace.HBM),
            pl.BlockSpec(memory_space=pltpu.MemorySpace.HBM),
        ],
        out_shape=(
            jax.ShapeDtypeStruct((M, D), W.dtype),
            jax.ShapeDtypeStruct((1, B), jnp.int32),
        ),
        scratch_shapes=[
            pltpu.VMEM((B, D), jnp.float32),
            pltpu.VMEM((M, D), jnp.float32),
            pltpu.VMEM((1, B), jnp.int32),
            pltpu.SemaphoreType.DMA,
            pltpu.SemaphoreType.DMA,
            pltpu.SemaphoreType.DMA,
            pltpu.SemaphoreType.DMA,
            pltpu.SemaphoreType.DMA,
        ],
    )(inputs, W)
    return wout


# speedup vs baseline: 1.1254x; 1.1254x over previous
import jax
import jax.numpy as jnp
from jax import lax
from jax.experimental import pallas as pl
from jax.experimental.pallas import tpu as pltpu


def _som_body(x_hbm, w_hbm, wout_hbm, winner_hbm,
              x_v, w_v, win_v, sem_x, sem_w, sem_out, sem_win):
    cp_x = pltpu.make_async_copy(x_hbm, x_v, sem_x)
    cp_w = pltpu.make_async_copy(w_hbm, w_v, sem_w)
    cp_w.start()
    cp_x.start()
    cp_w.wait()
    cp_out = pltpu.make_async_copy(w_v, wout_hbm, sem_out)
    cp_out.start()
    w = w_v[...]
    wnh = 0.5 * jnp.sum(w * w, axis=1, keepdims=True)
    cp_x.wait()
    x = x_v[...]
    xwt = lax.dot_general(w, x, (((1,), (1,)), ((), ())),
                          preferred_element_type=jnp.float32)
    score = xwt - wnh
    win_v[...] = jnp.argmax(score, axis=0).astype(jnp.int32)[None, :]
    cp_win = pltpu.make_async_copy(win_v, winner_hbm, sem_win)
    cp_win.start()
    cp_win.wait()
    cp_out.wait()


def kernel(inputs, W):
    B, D = inputs.shape
    M, _ = W.shape
    wout, _winner = pl.pallas_call(
        _som_body,
        in_specs=[
            pl.BlockSpec(memory_space=pltpu.MemorySpace.HBM),
            pl.BlockSpec(memory_space=pltpu.MemorySpace.HBM),
        ],
        out_specs=[
            pl.BlockSpec(memory_space=pltpu.MemorySpace.HBM),
            pl.BlockSpec(memory_space=pltpu.MemorySpace.HBM),
        ],
        out_shape=(
            jax.ShapeDtypeStruct((M, D), W.dtype),
            jax.ShapeDtypeStruct((1, B), jnp.int32),
        ),
        scratch_shapes=[
            pltpu.VMEM((B, D), jnp.float32),
            pltpu.VMEM((M, D), jnp.float32),
            pltpu.VMEM((1, B), jnp.int32),
            pltpu.SemaphoreType.DMA,
            pltpu.SemaphoreType.DMA,
            pltpu.SemaphoreType.DMA,
            pltpu.SemaphoreType.DMA,
        ],
    )(inputs, W)
    return wout
